# Initial kernel scaffold; baseline (speedup 1.0000x reference)
import jax
import jax.numpy as jnp
from jax.experimental import pallas as pl


def kernel(elems, table):
    raise NotImplementedError("write your pallas kernel here")


# speedup vs baseline: 2.8238x; 2.8238x over previous
import functools

import jax
import jax.numpy as jnp
from jax import lax
from jax.experimental import pallas as pl
from jax.experimental.pallas import tpu as pltpu
from jax.experimental.pallas import tpu_sc as plsc

_CH = 128


@functools.lru_cache(maxsize=None)
def _make_gather(B, V, D, nc, ns):
    NW = nc * ns
    b_per_w = B // NW
    n_chunks = b_per_w // _CH
    mesh = plsc.VectorSubcoreMesh(core_axis_name="c", subcore_axis_name="s")

    @functools.partial(
        pl.kernel,
        mesh=mesh,
        out_type=jax.ShapeDtypeStruct((B, D), jnp.float32),
        scratch_types=[
            pltpu.VMEM((b_per_w,), jnp.int32),
            pltpu.VMEM((_CH, D), jnp.float32),
            pltpu.SemaphoreType.DMA,
        ],
    )
    def gather_kernel(idx_hbm, table_hbm, out_hbm, idx_v, rows_v, sem):
        wid = lax.axis_index("s") * nc + lax.axis_index("c")
        base = wid * b_per_w
        pltpu.sync_copy(idx_hbm.at[pl.ds(base, b_per_w)], idx_v)

        def body(g, carry):
            idx_sl = idx_v.at[pl.ds(g * _CH, _CH)]
            pltpu.async_copy(table_hbm.at[idx_sl], rows_v, sem).wait()
            pltpu.sync_copy(rows_v, out_hbm.at[pl.ds(base + g * _CH, _CH)])
            return carry

        lax.fori_loop(0, n_chunks, body, 0)

    return gather_kernel


def kernel(elems, table):
    shape = elems.shape
    V, D = table.shape
    idx = elems.reshape(-1).astype(jnp.int32)
    B = idx.shape[0]
    info = plsc.get_sparse_core_info()
    nc, ns = info.num_cores, info.num_subcores
    group = nc * ns * _CH
    Bp = ((B + group - 1) // group) * group
    if Bp != B:
        idx = jnp.pad(idx, (0, Bp - B))
    out = _make_gather(Bp, V, D, nc, ns)(idx, table)
    if Bp != B:
        out = out[:B]
    return out.reshape(*shape, D)


# speedup vs baseline: 2.8770x; 1.0188x over previous
import functools

import jax
import jax.numpy as jnp
from jax import lax
from jax.experimental import pallas as pl
from jax.experimental.pallas import tpu as pltpu
from jax.experimental.pallas import tpu_sc as plsc

_CH = 128
_NBUF = 4


@functools.lru_cache(maxsize=None)
def _make_gather(B, V, D, nc, ns):
    NW = nc * ns
    b_per_w = B // NW
    n_chunks = b_per_w // _CH
    assert n_chunks % _NBUF == 0
    mesh = plsc.VectorSubcoreMesh(core_axis_name="c", subcore_axis_name="s")

    @functools.partial(
        pl.kernel,
        mesh=mesh,
        out_type=jax.ShapeDtypeStruct((B, D), jnp.float32),
        scratch_types=[
            pltpu.VMEM((b_per_w,), jnp.int32),
            pltpu.VMEM((_NBUF, _CH, D), jnp.float32),
        ]
        + [pltpu.SemaphoreType.DMA] * (2 * _NBUF),
    )
    def gather_kernel(idx_hbm, table_hbm, out_hbm, idx_v, rows_v, *sems):
        sem_g = sems[:_NBUF]
        sem_s = sems[_NBUF:]
        wid = lax.axis_index("s") * nc + lax.axis_index("c")
        base = wid * b_per_w
        pltpu.sync_copy(idx_hbm.at[pl.ds(base, b_per_w)], idx_v)

        def gather_desc(g, b):
            idx_sl = idx_v.at[pl.ds(g * _CH, _CH)]
            return pltpu.make_async_copy(
                table_hbm.at[idx_sl], rows_v.at[b], sem_g[b])

        def scatter_desc(g, b):
            return pltpu.make_async_copy(
                rows_v.at[b], out_hbm.at[pl.ds(base + g * _CH, _CH)], sem_s[b])

        gather_desc(0, 0).start()
        gather_desc(1, 1).start()

        def body(gg, carry):
            for b in range(_NBUF):
                g = gg * _NBUF + b
                bg = (b + 2) % _NBUF

                @pl.when(g >= 2)
                def _():
                    scatter_desc(g - 2, bg).wait()

                @pl.when(g + 2 < n_chunks)
                def _():
                    gather_desc(g + 2, bg).start()

                gather_desc(g, b).wait()
                scatter_desc(g, b).start()
            return carry

        lax.fori_loop(0, n_chunks // _NBUF, body, 0)
        scatter_desc(n_chunks - 2, (n_chunks - 2) % _NBUF).wait()
        scatter_desc(n_chunks - 1, (n_chunks - 1) % _NBUF).wait()

    return gather_kernel


def kernel(elems, table):
    shape = elems.shape
    V, D = table.shape
    idx = elems.reshape(-1).astype(jnp.int32)
    B = idx.shape[0]
    info = plsc.get_sparse_core_info()
    nc, ns = info.num_cores, info.num_subcores
    group = nc * ns * _CH * _NBUF
    Bp = ((B + group - 1) // group) * group
    if Bp != B:
        idx = jnp.pad(idx, (0, Bp - B))
    out = _make_gather(Bp, V, D, nc, ns)(idx, table)
    if Bp != B:
        out = out[:B]
    return out.reshape(*shape, D)


# speedup vs baseline: 3.2601x; 1.1332x over previous
import functools

import jax
import jax.numpy as jnp
from jax import lax
from jax.experimental import pallas as pl
from jax.experimental.pallas import tpu as pltpu
from jax.experimental.pallas import tpu_sc as plsc

_CH = 128
_NBUF = 4
_RUNROLL = 16
_L = 16


@functools.lru_cache(maxsize=None)
def _make_lookup(B, V, D, nc, ns):
    NW = nc * ns
    b_per_w = B // NW
    n_chunks = b_per_w // _CH
    assert n_chunks % _NBUF == 0 and _CH % _RUNROLL == 0 and D % _L == 0
    mesh = plsc.VectorSubcoreMesh(core_axis_name="c", subcore_axis_name="s")

    @functools.partial(
        pl.kernel,
        mesh=mesh,
        out_type=jax.ShapeDtypeStruct((B * D,), jnp.float32),
        scratch_types=[
            pltpu.VMEM((V * D,), jnp.float32),
            pltpu.VMEM((b_per_w,), jnp.int32),
            pltpu.VMEM((_NBUF, _CH * D), jnp.float32),
        ]
        + [pltpu.SemaphoreType.DMA] * _NBUF,
    )
    def lookup_kernel(idx_hbm, table_hbm, out_hbm, table_v, idx_v, rows_v,
                      *sem_s):
        wid = lax.axis_index("s") * nc + lax.axis_index("c")
        base = wid * b_per_w
        pltpu.sync_copy(table_hbm, table_v)
        pltpu.sync_copy(idx_hbm.at[pl.ds(base, b_per_w)], idx_v)

        def scatter_desc(g, b):
            return pltpu.make_async_copy(
                rows_v.at[b],
                out_hbm.at[pl.ds((base + g * _CH) * D, _CH * D)],
                sem_s[b])

        def body(gg, carry):
            for b in range(_NBUF):
                g = gg * _NBUF + b

                @pl.when(g >= _NBUF)
                def _():
                    scatter_desc(g - _NBUF, b).wait()

                rows_b = rows_v.at[b]

                def row_body(rr, c):
                    ivec = idx_v[pl.ds(g * _CH + rr * _RUNROLL, _L)] * D
                    for u in range(_RUNROLL):
                        r = rr * _RUNROLL + u
                        src = ivec[u]
                        for d in range(D // _L):
                            rows_b[pl.ds(r * D + d * _L, _L)] = (
                                table_v[pl.ds(src + d * _L, _L)])
                    return c

                lax.fori_loop(0, _CH // _RUNROLL, row_body, 0)
                scatter_desc(g, b).start()
            return carry

        lax.fori_loop(0, n_chunks // _NBUF, body, 0)
        for b in range(_NBUF):
            scatter_desc(n_chunks - _NBUF + b, b).wait()

    return lookup_kernel


def kernel(elems, table):
    shape = elems.shape
    V, D = table.shape
    idx = elems.reshape(-1).astype(jnp.int32)
    B = idx.shape[0]
    info = plsc.get_sparse_core_info()
    nc, ns = info.num_cores, info.num_subcores
    group = nc * ns * _CH * _NBUF
    Bp = ((B + group - 1) // group) * group
    if Bp != B:
        idx = jnp.pad(idx, (0, Bp - B))
    out = _make_lookup(Bp, V, D, nc, ns)(idx, table.reshape(-1))
    out = out.reshape(Bp, D)
    if Bp != B:
        out = out[:B]
    return out.reshape(*shape, D)


# speedup vs baseline: 4.3104x; 1.3221x over previous
import functools

import jax
import jax.numpy as jnp
from jax import lax
from jax.experimental import pallas as pl
from jax.experimental.pallas import tpu as pltpu
from jax.experimental.pallas import tpu_sc as plsc

_CH = 128
_NBUF = 4
_RUNROLL = 16
_L = 16


@functools.lru_cache(maxsize=None)
def _make_lookup(B, V, D, nc, ns):
    NW = nc * ns
    b_per_w = B // NW
    n_chunks = b_per_w // _CH
    assert n_chunks % _NBUF == 0 and _CH % _RUNROLL == 0 and D % _L == 0
    mesh = plsc.VectorSubcoreMesh(core_axis_name="c", subcore_axis_name="s")

    @functools.partial(
        pl.kernel,
        mesh=mesh,
        out_type=jax.ShapeDtypeStruct((B, D), jnp.float32),
        scratch_types=[
            pltpu.VMEM((V, D), jnp.float32),
            pltpu.VMEM((b_per_w,), jnp.int32),
            pltpu.VMEM((_NBUF, _CH, D), jnp.float32),
        ]
        + [pltpu.SemaphoreType.DMA] * (2 + _NBUF),
    )
    def lookup_kernel(idx_hbm, table_hbm, out_hbm, table_v, idx_v, rows_v,
                      *sems):
        sem_g = sems[:2]
        sem_s = sems[2:]
        wid = lax.axis_index("s") * nc + lax.axis_index("c")
        base = wid * b_per_w
        pltpu.sync_copy(table_hbm, table_v)
        pltpu.sync_copy(idx_hbm.at[pl.ds(base, b_per_w)], idx_v)

        def gather_desc(g, b):
            idx_sl = idx_v.at[pl.ds(g * _CH, _CH)]
            return pltpu.make_async_copy(
                table_hbm.at[idx_sl], rows_v.at[b], sem_g[b])

        def scatter_desc(g, b):
            return pltpu.make_async_copy(
                rows_v.at[b],
                out_hbm.at[pl.ds(base + g * _CH, _CH)],
                sem_s[b])

        def copy_chunk(g, b):
            rows_b = rows_v.at[b]

            def row_body(rr, c):
                ivec = idx_v[pl.ds(g * _CH + rr * _RUNROLL, _L)]
                for u in range(_RUNROLL):
                    r = rr * _RUNROLL + u
                    src_row = table_v.at[ivec[u]]
                    dst_row = rows_b.at[r]
                    for d in range(D // _L):
                        dst_row[pl.ds(d * _L, _L)] = src_row[pl.ds(d * _L, _L)]
                return c

            lax.fori_loop(0, _CH // _RUNROLL, row_body, 0)

        def body(gg, carry):
            g0 = gg * _NBUF
            for b in range(2):
                @pl.when(gg > 0)
                def _():
                    scatter_desc(g0 + b - _NBUF, b).wait()
                gather_desc(g0 + b, b).start()
            for b in range(2, _NBUF):
                @pl.when(gg > 0)
                def _():
                    scatter_desc(g0 + b - _NBUF, b).wait()
                copy_chunk(g0 + b, b)
                scatter_desc(g0 + b, b).start()
            for b in range(2):
                gather_desc(g0 + b, b).wait()
                scatter_desc(g0 + b, b).start()
            return carry

        lax.fori_loop(0, n_chunks // _NBUF, body, 0)
        for b in range(_NBUF):
            scatter_desc(n_chunks - _NBUF + b, b).wait()

    return lookup_kernel


def kernel(elems, table):
    shape = elems.shape
    V, D = table.shape
    idx = elems.reshape(-1).astype(jnp.int32)
    B = idx.shape[0]
    info = plsc.get_sparse_core_info()
    nc, ns = info.num_cores, info.num_subcores
    group = nc * ns * _CH * _NBUF
    Bp = ((B + group - 1) // group) * group
    if Bp != B:
        idx = jnp.pad(idx, (0, Bp - B))
    out = _make_lookup(Bp, V, D, nc, ns)(idx, table)
    if Bp != B:
        out = out[:B]
    return out.reshape(*shape, D)


# speedup vs baseline: 6.0842x; 1.4115x over previous
import functools

import jax
import jax.numpy as jnp
from jax import lax
from jax.experimental import pallas as pl
from jax.experimental.pallas import tpu as pltpu
from jax.experimental.pallas import tpu_sc as plsc

_CH = 128
_NBUF = 4
_RUNROLL = 16
_L = 16


@functools.lru_cache(maxsize=None)
def _make_lookup(B, V, D, nc, ns):
    NW = nc * ns
    b_per_w = B // NW
    n_chunks = b_per_w // _CH
    assert n_chunks % _NBUF == 0 and _CH % _RUNROLL == 0 and D % _L == 0
    mesh = plsc.VectorSubcoreMesh(core_axis_name="c", subcore_axis_name="s")

    @functools.partial(
        pl.kernel,
        mesh=mesh,
        out_type=jax.ShapeDtypeStruct((B, D), jnp.float32),
        scratch_types=[
            pltpu.VMEM((V, D), jnp.float32),
            pltpu.VMEM_SHARED((V, D), jnp.float32),
            pltpu.VMEM((b_per_w,), jnp.int32),
            pltpu.VMEM((_NBUF, _CH, D), jnp.float32),
        ]
        + [pltpu.SemaphoreType.DMA] * (2 + _NBUF),
    )
    def lookup_kernel(idx_hbm, table_hbm, out_hbm, table_v, table_sh, idx_v,
                      rows_v, *sems):
        sem_g = sems[:2]
        sem_s = sems[2:]
        wid = lax.axis_index("s") * nc + lax.axis_index("c")
        base = wid * b_per_w
        pltpu.sync_copy(table_hbm, table_v)

        @pl.when(lax.axis_index("s") == 0)
        def _():
            pltpu.sync_copy(table_hbm, table_sh)

        pltpu.sync_copy(idx_hbm.at[pl.ds(base, b_per_w)], idx_v)
        plsc.subcore_barrier()

        def gather_desc(g, b):
            idx_sl = idx_v.at[pl.ds(g * _CH, _CH)]
            return pltpu.make_async_copy(
                table_sh.at[idx_sl], rows_v.at[b], sem_g[b])

        def scatter_desc(g, b):
            return pltpu.make_async_copy(
                rows_v.at[b],
                out_hbm.at[pl.ds(base + g * _CH, _CH)],
                sem_s[b])

        def copy_chunk(g, b):
            rows_b = rows_v.at[b]

            def row_body(rr, c):
                ivec = idx_v[pl.ds(g * _CH + rr * _RUNROLL, _L)]
                for u in range(_RUNROLL):
                    r = rr * _RUNROLL + u
                    src_row = table_v.at[ivec[u]]
                    dst_row = rows_b.at[r]
                    for d in range(D // _L):
                        dst_row[pl.ds(d * _L, _L)] = src_row[pl.ds(d * _L, _L)]
                return c

            lax.fori_loop(0, _CH // _RUNROLL, row_body, 0)

        def body(gg, carry):
            g0 = gg * _NBUF
            for b in range(2):
                @pl.when(gg > 0)
                def _():
                    scatter_desc(g0 + b - _NBUF, b).wait()
                gather_desc(g0 + b, b).start()
            for b in range(2, _NBUF):
                @pl.when(gg > 0)
                def _():
                    scatter_desc(g0 + b - _NBUF, b).wait()
                copy_chunk(g0 + b, b)
                scatter_desc(g0 + b, b).start()
            for b in range(2):
                gather_desc(g0 + b, b).wait()
                scatter_desc(g0 + b, b).start()
            return carry

        lax.fori_loop(0, n_chunks // _NBUF, body, 0)
        for b in range(_NBUF):
            scatter_desc(n_chunks - _NBUF + b, b).wait()

    return lookup_kernel


def kernel(elems, table):
    shape = elems.shape
    V, D = table.shape
    idx = elems.reshape(-1).astype(jnp.int32)
    B = idx.shape[0]
    info = plsc.get_sparse_core_info()
    nc, ns = info.num_cores, info.num_subcores
    group = nc * ns * _CH * _NBUF
    Bp = ((B + group - 1) // group) * group
    if Bp != B:
        idx = jnp.pad(idx, (0, Bp - B))
    out = _make_lookup(Bp, V, D, nc, ns)(idx, table)
    if Bp != B:
        out = out[:B]
    return out.reshape(*shape, D)


# speedup vs baseline: 15.3713x; 2.5264x over previous
import functools

import jax
import jax.numpy as jnp
from jax import lax
from jax.experimental import pallas as pl
from jax.experimental.pallas import tpu as pltpu
from jax.experimental.pallas import tpu_sc as plsc

_CH = 128
_NBUF = 4
_RUNROLL = 16
_L = 16


@functools.lru_cache(maxsize=None)
def _make_lookup(B, V, D, nc, ns):
    NW = nc * ns
    b_per_w = B // NW
    n_chunks = b_per_w // _CH
    assert n_chunks % _NBUF == 0 and _CH % _RUNROLL == 0 and D % _L == 0
    mesh = plsc.VectorSubcoreMesh(core_axis_name="c", subcore_axis_name="s")

    @functools.partial(
        pl.kernel,
        mesh=mesh,
        out_type=jax.ShapeDtypeStruct((B, D), jnp.float32),
        scratch_types=[
            pltpu.VMEM((V, D), jnp.float32),
            pltpu.VMEM_SHARED((V, D), jnp.float32),
            pltpu.VMEM((b_per_w,), jnp.int32),
            pltpu.VMEM((_NBUF, _CH, D), jnp.float32),
        ]
        + [pltpu.SemaphoreType.DMA] * (2 * _NBUF),
    )
    def lookup_kernel(idx_hbm, table_hbm, out_hbm, table_v, table_sh, idx_v,
                      rows_v, *sems):
        sem_g = sems[:_NBUF]
        sem_s = sems[_NBUF:]
        wid = lax.axis_index("s") * nc + lax.axis_index("c")
        base = wid * b_per_w
        pltpu.sync_copy(table_hbm, table_v)

        @pl.when(lax.axis_index("s") == 0)
        def _():
            pltpu.sync_copy(table_hbm, table_sh)

        pltpu.sync_copy(idx_hbm.at[pl.ds(base, b_per_w)], idx_v)
        plsc.subcore_barrier()

        def gather_desc(g, b):
            idx_sl = idx_v.at[pl.ds(g * _CH, _CH)]
            return pltpu.make_async_copy(
                table_sh.at[idx_sl], rows_v.at[b], sem_g[b])

        def scatter_desc(g, b):
            return pltpu.make_async_copy(
                rows_v.at[b],
                out_hbm.at[pl.ds(base + g * _CH, _CH)],
                sem_s[b])

        def copy_chunk(g, b):
            rows_b = rows_v.at[b]

            def row_body(rr, c):
                ivec = idx_v[pl.ds(g * _CH + rr * _RUNROLL, _L)]
                for u in range(_RUNROLL):
                    r = rr * _RUNROLL + u
                    src_row = table_v.at[ivec[u]]
                    dst_row = rows_b.at[r]
                    for d in range(D // _L):
                        dst_row[pl.ds(d * _L, _L)] = src_row[pl.ds(d * _L, _L)]
                return c

            lax.fori_loop(0, _CH // _RUNROLL, row_body, 0)

        def body(gg, carry):
            g0 = gg * _NBUF
            for b in range(_NBUF):
                @pl.when(gg > 0)
                def _():
                    scatter_desc(g0 + b - _NBUF, b).wait()
                gather_desc(g0 + b, b).start()
            for b in range(_NBUF):
                gather_desc(g0 + b, b).wait()
                scatter_desc(g0 + b, b).start()
            return carry

        lax.fori_loop(0, n_chunks // _NBUF, body, 0)
        for b in range(_NBUF):
            scatter_desc(n_chunks - _NBUF + b, b).wait()

    return lookup_kernel


def kernel(elems, table):
    shape = elems.shape
    V, D = table.shape
    idx = elems.reshape(-1).astype(jnp.int32)
    B = idx.shape[0]
    info = plsc.get_sparse_core_info()
    nc, ns = info.num_cores, info.num_subcores
    group = nc * ns * _CH * _NBUF
    Bp = ((B + group - 1) // group) * group
    if Bp != B:
        idx = jnp.pad(idx, (0, Bp - B))
    out = _make_lookup(Bp, V, D, nc, ns)(idx, table)
    if Bp != B:
        out = out[:B]
    return out.reshape(*shape, D)


# speedup vs baseline: 15.9956x; 1.0406x over previous
import functools

import jax
import jax.numpy as jnp
from jax import lax
from jax.experimental import pallas as pl
from jax.experimental.pallas import tpu as pltpu
from jax.experimental.pallas import tpu_sc as plsc

_CH = 128
_NBUF = 4


@functools.lru_cache(maxsize=None)
def _make_lookup(B, V, D, nc, ns):
    NW = nc * ns
    b_per_w = B // NW
    n_chunks = b_per_w // _CH
    assert n_chunks % _NBUF == 0 and n_chunks >= _NBUF
    mesh = plsc.VectorSubcoreMesh(core_axis_name="c", subcore_axis_name="s")

    @functools.partial(
        pl.kernel,
        mesh=mesh,
        out_type=jax.ShapeDtypeStruct((B, D), jnp.float32),
        scratch_types=[
            pltpu.VMEM_SHARED((V, D), jnp.float32),
            pltpu.VMEM((b_per_w,), jnp.int32),
            pltpu.VMEM((_NBUF, _CH, D), jnp.float32),
        ]
        + [pltpu.SemaphoreType.DMA] * (2 * _NBUF),
    )
    def lookup_kernel(idx_hbm, table_hbm, out_hbm, table_sh, idx_v, rows_v,
                      *sems):
        sem_g = sems[:_NBUF]
        sem_s = sems[_NBUF:]
        wid = lax.axis_index("s") * nc + lax.axis_index("c")
        base = wid * b_per_w

        @pl.when(lax.axis_index("s") == 0)
        def _():
            pltpu.sync_copy(table_hbm, table_sh)

        pltpu.sync_copy(idx_hbm.at[pl.ds(base, b_per_w)], idx_v)
        plsc.subcore_barrier()

        def gather_desc(g, b):
            idx_sl = idx_v.at[pl.ds(g * _CH, _CH)]
            return pltpu.make_async_copy(
                table_sh.at[idx_sl], rows_v.at[b], sem_g[b])

        def scatter_desc(g, b):
            return pltpu.make_async_copy(
                rows_v.at[b],
                out_hbm.at[pl.ds(base + g * _CH, _CH)],
                sem_s[b])

        gather_desc(0, 0).start()
        gather_desc(1, 1).start()

        def body(gg, carry):
            for b in range(_NBUF):
                g = gg * _NBUF + b
                bg = (b + 2) % _NBUF

                @pl.when(g >= 2)
                def _():
                    scatter_desc(g - 2, bg).wait()

                @pl.when(g + 2 < n_chunks)
                def _():
                    gather_desc(g + 2, bg).start()

                gather_desc(g, b).wait()
                scatter_desc(g, b).start()
            return carry

        lax.fori_loop(0, n_chunks // _NBUF, body, 0)
        scatter_desc(n_chunks - 2, (n_chunks - 2) % _NBUF).wait()
        scatter_desc(n_chunks - 1, (n_chunks - 1) % _NBUF).wait()

    return lookup_kernel


def kernel(elems, table):
    shape = elems.shape
    V, D = table.shape
    idx = elems.reshape(-1).astype(jnp.int32)
    B = idx.shape[0]
    info = plsc.get_sparse_core_info()
    nc, ns = info.num_cores, info.num_subcores
    group = nc * ns * _CH * _NBUF
    Bp = ((B + group - 1) // group) * group
    if Bp != B:
        idx = jnp.pad(idx, (0, Bp - B))
    out = _make_lookup(Bp, V, D, nc, ns)(idx, table)
    if Bp != B:
        out = out[:B]
    return out.reshape(*shape, D)
